# Initial kernel scaffold; baseline (speedup 1.0000x reference)
import jax
import jax.numpy as jnp
from jax.experimental import pallas as pl


def kernel(table, W1, b1, W2, b2, sparse_ids, seq_ids, target_id, mask):
    raise NotImplementedError("write your pallas kernel here")


# speedup vs baseline: 1.6281x; 1.6281x over previous
import functools

import jax
import jax.numpy as jnp
from jax import lax
from jax.experimental import pallas as pl
from jax.experimental.pallas import tpu as pltpu
from jax.experimental.pallas import tpu_sc as plsc

_NC = 2
_NS = 16
_NW = _NC * _NS
_CH = 128


def _sc_gather(table, sp, sq, tg, B, F, L, D):
    nf = (B * F) // (_NW * _CH)
    ns = (B * L) // (_NW * _CH)
    nt = B // (_NW * _CH)
    ntot = nf + ns + nt
    mesh = plsc.VectorSubcoreMesh(core_axis_name="c", subcore_axis_name="s")

    @functools.partial(
        pl.kernel,
        out_type=(
            jax.ShapeDtypeStruct((B * F, D), jnp.float32),
            jax.ShapeDtypeStruct((L * B, D), jnp.float32),
            jax.ShapeDtypeStruct((B, D), jnp.float32),
        ),
        mesh=mesh,
        compiler_params=pltpu.CompilerParams(use_tc_tiling_on_sc=False),
        scratch_types=[
            pltpu.VMEM((ntot, _CH), jnp.int32),
            pltpu.VMEM((_CH, D), jnp.float32),
            pltpu.SemaphoreType.DMA,
        ],
    )
    def k(table_h, sp_h, sq_h, tg_h, of_h, ok_h, oq_h, idx_v, rows_v, gsem):
        wid = lax.axis_index("s") * _NC + lax.axis_index("c")
        pltpu.sync_copy(sp_h.at[pl.ds(wid * nf, nf)], idx_v.at[pl.ds(0, nf)])
        pltpu.sync_copy(sq_h.at[pl.ds(wid * ns, ns)], idx_v.at[pl.ds(nf, ns)])
        pltpu.sync_copy(tg_h.at[pl.ds(wid * nt, nt)], idx_v.at[pl.ds(nf + ns, nt)])

        def section(out_h, idx_base, out_base, nch):
            def body(j, carry):
                pltpu.async_copy(
                    table_h.at[idx_v.at[idx_base + j]], rows_v, gsem
                ).wait()
                pltpu.sync_copy(rows_v, out_h.at[pl.ds(out_base + j * _CH, _CH)])
                return carry

            lax.fori_loop(0, nch, body, 0)

        section(of_h, 0, wid * nf * _CH, nf)
        section(ok_h, nf, wid * ns * _CH, ns)
        section(oq_h, nf + ns, wid * nt * _CH, nt)

    return k(table, sp, sq, tg)


def _tc_att(q, keys3, mask3, W1, b1r, W2t, b2r, B, L, D, units, Bb):
    nb = B // Bb

    def body(q_ref, k_ref, m_ref, w1_ref, b1_ref, w2_ref, b2_ref, o_ref):
        l = pl.program_id(1)
        qv = q_ref[...]
        kv = k_ref[0]
        x = jnp.concatenate([qv, kv, qv * kv], axis=1)
        h = jnp.dot(x, w1_ref[...], preferred_element_type=jnp.float32)
        h = jnp.maximum(h + b1_ref[...], 0.0)
        s = jnp.sum(h * w2_ref[...], axis=1, keepdims=True) + b2_ref[...]
        s = s * m_ref[0]
        contrib = s * kv

        @pl.when(l == 0)
        def _():
            o_ref[...] = contrib

        @pl.when(l > 0)
        def _():
            o_ref[...] = o_ref[...] + contrib

    return pl.pallas_call(
        body,
        grid=(nb, L),
        in_specs=[
            pl.BlockSpec((Bb, D), lambda i, l: (i, 0)),
            pl.BlockSpec((1, Bb, D), lambda i, l: (l, i, 0)),
            pl.BlockSpec((1, Bb, 1), lambda i, l: (l, i, 0)),
            pl.BlockSpec((3 * D, units), lambda i, l: (0, 0)),
            pl.BlockSpec((1, units), lambda i, l: (0, 0)),
            pl.BlockSpec((1, units), lambda i, l: (0, 0)),
            pl.BlockSpec((1, 1), lambda i, l: (0, 0)),
        ],
        out_specs=pl.BlockSpec((Bb, D), lambda i, l: (i, 0)),
        out_shape=jax.ShapeDtypeStruct((B, D), jnp.float32),
    )(q, keys3, mask3, W1, b1r, W2t, b2r)


def kernel(table, W1, b1, W2, b2, sparse_ids, seq_ids, target_id, mask):
    B, F = sparse_ids.shape
    L = seq_ids.shape[1]
    D = table.shape[1]
    units = W1.shape[1]

    sp = sparse_ids.astype(jnp.int32).reshape(-1, _CH)
    sq = seq_ids.astype(jnp.int32).T.reshape(-1, _CH)
    tg = target_id.astype(jnp.int32).reshape(-1, _CH)

    field_rows, keys_lb, query = _sc_gather(table, sp, sq, tg, B, F, L, D)

    keys3 = keys_lb.reshape(L, B, D)
    mask3 = mask.T.reshape(L, B, 1)
    att = _tc_att(query, keys3, mask3, W1, b1.reshape(1, units),
                  W2.reshape(units, 1).T, b2.reshape(1, 1),
                  B, L, D, units, Bb=2048)

    return jnp.concatenate([field_rows.reshape(B, F * D), att], axis=1)


# speedup vs baseline: 2.2185x; 1.3627x over previous
import functools

import jax
import jax.numpy as jnp
from jax import lax
from jax.experimental import pallas as pl
from jax.experimental.pallas import tpu as pltpu
from jax.experimental.pallas import tpu_sc as plsc

_NC = 2
_NS = 16
_NW = _NC * _NS
_CH = 128
_G = 8


def _sc_gather(table, sp, sq, tg, B, F, L, D):
    nf = (B * F) // (_NW * _CH)
    ns = (B * L) // (_NW * _CH)
    nt = B // (_NW * _CH)
    ntot = nf + ns + nt
    assert nf % _G == 0 and ns % _G == 0 and nf // _G >= 2 and ns // _G >= 2
    mesh = plsc.VectorSubcoreMesh(core_axis_name="c", subcore_axis_name="s")

    @functools.partial(
        pl.kernel,
        out_type=(
            jax.ShapeDtypeStruct((B * F, D), jnp.float32),
            jax.ShapeDtypeStruct((L * B, D), jnp.float32),
            jax.ShapeDtypeStruct((B, D), jnp.float32),
        ),
        mesh=mesh,
        compiler_params=pltpu.CompilerParams(use_tc_tiling_on_sc=False),
        scratch_types=[
            pltpu.VMEM((ntot, _CH), jnp.int32),
            pltpu.VMEM((2, _G, _CH, D), jnp.float32),
            pltpu.SemaphoreType.DMA,
            pltpu.SemaphoreType.DMA,
        ],
    )
    def k(table_h, sp_h, sq_h, tg_h, of_h, ok_h, oq_h, idx_v, rows_v, gsem, ssem):
        wid = lax.axis_index("s") * _NC + lax.axis_index("c")
        pltpu.sync_copy(sp_h.at[pl.ds(wid * nf, nf)], idx_v.at[pl.ds(0, nf)])
        pltpu.sync_copy(sq_h.at[pl.ds(wid * ns, ns)], idx_v.at[pl.ds(nf, ns)])
        pltpu.sync_copy(tg_h.at[pl.ds(wid * nt, nt)], idx_v.at[pl.ds(nf + ns, nt)])

        def drain_one_scatter(out_h):
            pltpu.make_async_copy(
                out_h.at[pl.ds(0, _CH)], rows_v.at[0, 0], ssem
            ).wait()

        def section(out_h, idx_base, out_base, nch):
            ng = nch // _G

            def body(g, carry):
                s = lax.rem(g, 2)

                @pl.when(g >= 2)
                def _():
                    for _ in range(_G):
                        drain_one_scatter(out_h)

                descs = []
                for b in range(_G):
                    descs.append(
                        pltpu.async_copy(
                            table_h.at[idx_v.at[idx_base + g * _G + b]],
                            rows_v.at[s, b],
                            gsem,
                        )
                    )
                for dsc in descs:
                    dsc.wait()
                for b in range(_G):
                    pltpu.async_copy(
                        rows_v.at[s, b],
                        out_h.at[pl.ds(out_base + (g * _G + b) * _CH, _CH)],
                        ssem,
                    )
                return carry

            lax.fori_loop(0, ng, body, 0)
            for _ in range(2 * _G):
                drain_one_scatter(out_h)

        section(of_h, 0, wid * nf * _CH, nf)
        section(ok_h, nf, wid * ns * _CH, ns)
        for j in range(nt):
            pltpu.async_copy(
                table_h.at[idx_v.at[nf + ns + j]], rows_v.at[0, 0], gsem
            ).wait()
            pltpu.sync_copy(
                rows_v.at[0, 0], oq_h.at[pl.ds((wid * nt + j) * _CH, _CH)]
            )

    return k(table, sp, sq, tg)


def _tc_att(qt, keys_t, mask_t, w1t, b1c, w2t, b2c, B, L, D, units, Bb):
    nb = B // Bb

    def body(q_ref, k_ref, m_ref, w1_ref, b1_ref, w2_ref, b2_ref, o_ref):
        l = pl.program_id(1)
        qv = q_ref[...]
        kv = k_ref[0]
        x = jnp.concatenate([qv, kv, qv * kv], axis=0)
        h = jnp.dot(w1_ref[...], x, preferred_element_type=jnp.float32)
        h = jnp.maximum(h + b1_ref[...], 0.0)
        s = jnp.dot(w2_ref[...], h, preferred_element_type=jnp.float32)
        s = (s + b2_ref[...]) * m_ref[0]
        contrib = s * kv

        @pl.when(l == 0)
        def _():
            o_ref[...] = contrib

        @pl.when(l > 0)
        def _():
            o_ref[...] = o_ref[...] + contrib

    return pl.pallas_call(
        body,
        grid=(nb, L),
        in_specs=[
            pl.BlockSpec((D, Bb), lambda i, l: (0, i)),
            pl.BlockSpec((1, D, Bb), lambda i, l: (l, 0, i)),
            pl.BlockSpec((1, 1, Bb), lambda i, l: (l, 0, i)),
            pl.BlockSpec((units, 3 * D), lambda i, l: (0, 0)),
            pl.BlockSpec((units, 1), lambda i, l: (0, 0)),
            pl.BlockSpec((1, units), lambda i, l: (0, 0)),
            pl.BlockSpec((1, 1), lambda i, l: (0, 0)),
        ],
        out_specs=pl.BlockSpec((D, Bb), lambda i, l: (0, i)),
        out_shape=jax.ShapeDtypeStruct((D, B), jnp.float32),
    )(qt, keys_t, mask_t, w1t, b1c, w2t, b2c)


def kernel(table, W1, b1, W2, b2, sparse_ids, seq_ids, target_id, mask):
    B, F = sparse_ids.shape
    L = seq_ids.shape[1]
    D = table.shape[1]
    units = W1.shape[1]

    sp = sparse_ids.astype(jnp.int32).reshape(-1, _CH)
    sq = seq_ids.astype(jnp.int32).T.reshape(-1, _CH)
    tg = target_id.astype(jnp.int32).reshape(-1, _CH)

    field_rows, keys_lb, query = _sc_gather(table, sp, sq, tg, B, F, L, D)

    keys_t = jnp.transpose(keys_lb.reshape(L, B, D), (0, 2, 1))
    att_t = _tc_att(query.T, keys_t, mask.T.reshape(L, 1, B), W1.T,
                    b1.reshape(units, 1),
                    W2.reshape(units, 1).T, b2.reshape(1, 1),
                    B=B, L=L, D=D, units=units, Bb=2048)

    return jnp.concatenate([field_rows.reshape(B, F * D), att_t.T], axis=1)


# speedup vs baseline: 2.5504x; 1.1496x over previous
import functools

import jax
import jax.numpy as jnp
from jax import lax
from jax.experimental import pallas as pl
from jax.experimental.pallas import tpu as pltpu
from jax.experimental.pallas import tpu_sc as plsc

_NC = 2
_NS = 16
_NW = _NC * _NS
_CH = 128
_G = 8
_LC = 10


def _sc_gather_kernel(n_chunks, out_shapes):
    def build(table, idx_list, D):
        ntot = sum(n_chunks)
        mesh = plsc.VectorSubcoreMesh(core_axis_name="c", subcore_axis_name="s")

        @functools.partial(
            pl.kernel,
            out_type=tuple(
                jax.ShapeDtypeStruct((r, D), jnp.float32) for r in out_shapes
            ),
            mesh=mesh,
            compiler_params=pltpu.CompilerParams(use_tc_tiling_on_sc=False),
            scratch_types=[
                pltpu.VMEM((ntot, _CH), jnp.int32),
                pltpu.VMEM((2, _G, _CH, D), jnp.float32),
                pltpu.SemaphoreType.DMA,
                pltpu.SemaphoreType.DMA,
            ],
        )
        def k(table_h, *refs):
            nin = len(idx_list)
            idx_hs = refs[:nin]
            out_hs = refs[nin:2 * nin]
            idx_v, rows_v, gsem, ssem = refs[2 * nin:]
            wid = lax.axis_index("s") * _NC + lax.axis_index("c")

            base = 0
            for idx_h, nch in zip(idx_hs, n_chunks):
                pltpu.sync_copy(
                    idx_h.at[pl.ds(wid * nch, nch)], idx_v.at[pl.ds(base, nch)]
                )
                base += nch

            def drain_one_scatter(out_h):
                pltpu.make_async_copy(
                    out_h.at[pl.ds(0, _CH)], rows_v.at[0, 0], ssem
                ).wait()

            def section(out_h, idx_base, out_base, nch):
                if nch % _G != 0 or nch // _G < 2:
                    for j in range(nch):
                        pltpu.async_copy(
                            table_h.at[idx_v.at[idx_base + j]],
                            rows_v.at[0, 0], gsem,
                        ).wait()
                        pltpu.sync_copy(
                            rows_v.at[0, 0],
                            out_h.at[pl.ds(out_base + j * _CH, _CH)],
                        )
                    return
                ng = nch // _G

                def body(g, carry):
                    s = lax.rem(g, 2)

                    @pl.when(g >= 2)
                    def _():
                        for _ in range(_G):
                            drain_one_scatter(out_h)

                    descs = []
                    for b in range(_G):
                        descs.append(
                            pltpu.async_copy(
                                table_h.at[idx_v.at[idx_base + g * _G + b]],
                                rows_v.at[s, b],
                                gsem,
                            )
                        )
                    for dsc in descs:
                        dsc.wait()
                    for b in range(_G):
                        pltpu.async_copy(
                            rows_v.at[s, b],
                            out_h.at[pl.ds(out_base + (g * _G + b) * _CH, _CH)],
                            ssem,
                        )
                    return carry

                lax.fori_loop(0, ng, body, 0)
                for _ in range(2 * _G):
                    drain_one_scatter(out_h)

            idx_base = 0
            for out_h, nch in zip(out_hs, n_chunks):
                section(out_h, idx_base, wid * nch * _CH, nch)
                idx_base += nch

        return k(table, *idx_list)

    return build


def _tc_att(qt, keys_t, mask3, w1t, b1c, w2t, b2c, B, L, D, units, Bb):
    nb = B // Bb

    def body(q_ref, k_ref, m_ref, w1_ref, b1_ref, w2_ref, b2_ref, o_ref):
        l = pl.program_id(1)
        qv = q_ref[...]
        for j in range(_LC):
            kv = k_ref[j]
            x = jnp.concatenate([qv, kv, qv * kv], axis=0)
            h = jnp.dot(w1_ref[...], x, preferred_element_type=jnp.float32)
            h = jnp.maximum(h + b1_ref[...], 0.0)
            s = jnp.dot(w2_ref[...], h, preferred_element_type=jnp.float32)
            s = (s + b2_ref[...]) * m_ref[j]
            contrib = s * kv
            if j == 0:
                @pl.when(l == 0)
                def _():
                    o_ref[...] = contrib

                @pl.when(l > 0)
                def _():
                    o_ref[...] = o_ref[...] + contrib
            else:
                o_ref[...] = o_ref[...] + contrib

    return pl.pallas_call(
        body,
        grid=(nb, L // _LC),
        in_specs=[
            pl.BlockSpec((D, Bb), lambda i, l: (0, i)),
            pl.BlockSpec((_LC, D, Bb), lambda i, l: (l, 0, i)),
            pl.BlockSpec((_LC, 1, Bb), lambda i, l: (l, 0, i)),
            pl.BlockSpec((units, 3 * D), lambda i, l: (0, 0)),
            pl.BlockSpec((units, 1), lambda i, l: (0, 0)),
            pl.BlockSpec((1, units), lambda i, l: (0, 0)),
            pl.BlockSpec((1, 1), lambda i, l: (0, 0)),
        ],
        out_specs=pl.BlockSpec((D, Bb), lambda i, l: (0, i)),
        out_shape=jax.ShapeDtypeStruct((D, B), jnp.float32),
    )(qt, keys_t, mask3, w1t, b1c, w2t, b2c)


def kernel(table, W1, b1, W2, b2, sparse_ids, seq_ids, target_id, mask):
    B, F = sparse_ids.shape
    L = seq_ids.shape[1]
    D = table.shape[1]
    units = W1.shape[1]

    sp = sparse_ids.astype(jnp.int32).reshape(-1, _CH)
    sq = seq_ids.astype(jnp.int32).T.reshape(-1, _CH)
    tg = target_id.astype(jnp.int32).reshape(-1, _CH)

    ns = (B * L) // (_NW * _CH)
    nt = B // (_NW * _CH)
    nf = (B * F) // (_NW * _CH)

    keys_lb, query = _sc_gather_kernel([ns, nt], [L * B, B])(
        table, [sq, tg], D
    )
    (field_rows,) = _sc_gather_kernel([nf], [B * F])(table, [sp], D)

    keys_t = jnp.transpose(keys_lb.reshape(L, B, D), (0, 2, 1))
    att_t = _tc_att(query.T, keys_t, mask.T.reshape(L, 1, B), W1.T,
                    b1.reshape(units, 1),
                    W2.reshape(units, 1).T, b2.reshape(1, 1),
                    B=B, L=L, D=D, units=units, Bb=2048)

    return jnp.concatenate([field_rows.reshape(B, F * D), att_t.T], axis=1)
